# baseline (device time: 89960 ns/iter reference)
import jax
import jax.numpy as jnp
from jax import lax
from jax.experimental import pallas as pl
from jax.experimental.pallas import tpu as pltpu

N_DEV = 4
N_RS = N_DEV - 1
N_STEPS = 2 * N_RS
SUBS = 4


def kernel(A, B):
    m, k = A.shape
    _, n = B.shape
    half = m // 2
    ch = half // N_DEV
    sub_ch = ch // SUBS

    def send_idx(me, s):
        return (me - s) % N_DEV if s < N_RS else (me + 1 - (s - N_RS)) % N_DEV

    def recv_idx(me, s):
        return (me - s - 1) % N_DEV if s < N_RS else (me - (s - N_RS)) % N_DEV

    def body(a_ref, b_ref, out_ref, comm_r, comm_l, send_r, recv_r, send_l, recv_l):
        me = lax.axis_index("i")
        left = (me - 1) % N_DEV
        right = (me + 1) % N_DEV

        def top_row(c):
            return (c % N_DEV) * ch

        def bot_row(c):
            return half + (c % N_DEV) * ch

        def compute_chunk(row0):
            out_ref[pl.ds(row0, ch), :] = jnp.dot(
                a_ref[pl.ds(row0, ch), :], b_ref[:, :],
                preferred_element_type=jnp.float32,
            )

        def mk(s, sub, rows, dir_me, comm, send_sems, recv_sems, dst_dev):
            row_src = rows(send_idx(dir_me, s)) + sub * sub_ch
            src = out_ref.at[pl.ds(row_src, sub_ch), :]
            if s < N_RS:
                dst = comm.at[s, pl.ds(sub * sub_ch, sub_ch), :]
            else:
                dst = out_ref.at[pl.ds(row_src, sub_ch), :]
            return pltpu.make_async_remote_copy(
                src_ref=src,
                dst_ref=dst,
                send_sem=send_sems.at[s, sub],
                recv_sem=recv_sems.at[s, sub],
                device_id=(dst_dev,),
                device_id_type=pl.DeviceIdType.MESH,
            )

        def mk_r(s, sub):
            return mk(s, sub, top_row, me, comm_r, send_r, recv_r, right)

        def mk_l(s, sub):
            return mk(s, sub, bot_row, -me, comm_l, send_l, recv_l, left)

        barrier_sem = pltpu.get_barrier_semaphore()
        for nbr in (left, right):
            pl.semaphore_signal(
                barrier_sem, inc=1,
                device_id=(nbr,), device_id_type=pl.DeviceIdType.MESH,
            )
        compute_chunk(top_row(me))
        compute_chunk(bot_row(-me))
        pl.semaphore_wait(barrier_sem, 2)

        descs_r = [[None] * SUBS for _ in range(N_STEPS)]
        descs_l = [[None] * SUBS for _ in range(N_STEPS)]
        for sub in range(SUBS):
            descs_r[0][sub] = mk_r(0, sub)
            descs_r[0][sub].start()
            descs_l[0][sub] = mk_l(0, sub)
            descs_l[0][sub].start()
        for j in range(1, N_DEV):
            compute_chunk(top_row(me - j))
            compute_chunk(bot_row(-me - j))

        for s in range(N_STEPS):
            for sub in range(SUBS):
                for descs, mk_d, rows, dir_me, comm in (
                    (descs_r, mk_r, top_row, me, comm_r),
                    (descs_l, mk_l, bot_row, -me, comm_l),
                ):
                    descs[s][sub].wait()
                    if s < N_RS:
                        row = rows(recv_idx(dir_me, s)) + sub * sub_ch
                        out_ref[pl.ds(row, sub_ch), :] += comm[
                            s, sub * sub_ch:(sub + 1) * sub_ch, :
                        ]
                    if s + 1 < N_STEPS:
                        descs[s + 1][sub] = mk_d(s + 1, sub)
                        descs[s + 1][sub].start()

    return pl.pallas_call(
        body,
        out_shape=jax.ShapeDtypeStruct((m, n), jnp.float32),
        in_specs=[
            pl.BlockSpec(memory_space=pltpu.VMEM),
            pl.BlockSpec(memory_space=pltpu.VMEM),
        ],
        out_specs=pl.BlockSpec(memory_space=pltpu.VMEM),
        scratch_shapes=[
            pltpu.VMEM((N_RS, ch, n), jnp.float32),
            pltpu.VMEM((N_RS, ch, n), jnp.float32),
            pltpu.SemaphoreType.DMA((N_STEPS, SUBS)),
            pltpu.SemaphoreType.DMA((N_STEPS, SUBS)),
            pltpu.SemaphoreType.DMA((N_STEPS, SUBS)),
            pltpu.SemaphoreType.DMA((N_STEPS, SUBS)),
        ],
        compiler_params=pltpu.CompilerParams(collective_id=0),
    )(A, B)


# device time: 51702 ns/iter; 1.7400x vs baseline; 1.7400x over previous
import jax
import jax.numpy as jnp
from jax import lax
from jax.experimental import pallas as pl
from jax.experimental.pallas import tpu as pltpu

N_DEV = 4
N_RS = N_DEV - 1
N_STEPS = 2 * N_RS
SUBS = 2


def kernel(A, B):
    m, k = A.shape
    _, n = B.shape
    half = m // 2
    ch = half // N_DEV
    sub_ch = ch // SUBS

    def send_idx(me, s):
        return (me - s) % N_DEV if s < N_RS else (me + 1 - (s - N_RS)) % N_DEV

    def recv_idx(me, s):
        return (me - s - 1) % N_DEV if s < N_RS else (me - (s - N_RS)) % N_DEV

    def body(
        a_ref, b_ref, out_ref,
        comm_r, comm_l, stage_r, stage_l, ag_r, ag_l,
        send_r, recv_r, send_l, recv_l,
    ):
        me = lax.axis_index("i")
        left = (me - 1) % N_DEV
        right = (me + 1) % N_DEV

        def top_row(c):
            return (c % N_DEV) * ch

        def bot_row(c):
            return half + (c % N_DEV) * ch

        def compute_chunk(row0, stage=None):
            d = jnp.dot(
                a_ref[pl.ds(row0, ch), :], b_ref[:, :],
                preferred_element_type=jnp.float32,
            )
            out_ref[pl.ds(row0, ch), :] = d
            if stage is not None:
                stage[0, :, :] = d.astype(jnp.bfloat16)

        def mk(s, sub, stage, comm, ag, send_sems, recv_sems, dst_dev):
            rows = pl.ds(sub * sub_ch, sub_ch)
            src = stage.at[s, rows, :] if s <= N_RS else ag.at[s - N_RS - 1, rows, :]
            dst = comm.at[s, rows, :] if s < N_RS else ag.at[s - N_RS, rows, :]
            return pltpu.make_async_remote_copy(
                src_ref=src,
                dst_ref=dst,
                send_sem=send_sems.at[s, sub],
                recv_sem=recv_sems.at[s, sub],
                device_id=(dst_dev,),
                device_id_type=pl.DeviceIdType.MESH,
            )

        def mk_r(s, sub):
            return mk(s, sub, stage_r, comm_r, ag_r, send_r, recv_r, right)

        def mk_l(s, sub):
            return mk(s, sub, stage_l, comm_l, ag_l, send_l, recv_l, left)

        barrier_sem = pltpu.get_barrier_semaphore()
        for nbr in (left, right):
            pl.semaphore_signal(
                barrier_sem, inc=1,
                device_id=(nbr,), device_id_type=pl.DeviceIdType.MESH,
            )
        compute_chunk(top_row(me), stage_r)
        compute_chunk(bot_row(-me), stage_l)
        pl.semaphore_wait(barrier_sem, 2)

        descs_r = [[None] * SUBS for _ in range(N_STEPS)]
        descs_l = [[None] * SUBS for _ in range(N_STEPS)]
        for sub in range(SUBS):
            descs_r[0][sub] = mk_r(0, sub)
            descs_r[0][sub].start()
            descs_l[0][sub] = mk_l(0, sub)
            descs_l[0][sub].start()
        for j in range(1, N_DEV):
            compute_chunk(top_row(me - j))
            compute_chunk(bot_row(-me - j))

        for s in range(N_STEPS):
            for sub in range(SUBS):
                for descs, mk_d, rows_of, dir_me, comm, stage, ag in (
                    (descs_r, mk_r, top_row, me, comm_r, stage_r, ag_r),
                    (descs_l, mk_l, bot_row, -me, comm_l, stage_l, ag_l),
                ):
                    descs[s][sub].wait()
                    rows = pl.ds(rows_of(recv_idx(dir_me, s)) + sub * sub_ch, sub_ch)
                    srows = pl.ds(sub * sub_ch, sub_ch)
                    if s < N_RS:
                        acc = out_ref[rows, :] + comm[s, srows, :].astype(jnp.float32)
                        out_ref[rows, :] = acc
                        stage[s + 1, srows, :] = acc.astype(jnp.bfloat16)
                        descs[s + 1][sub] = mk_d(s + 1, sub)
                        descs[s + 1][sub].start()
                    else:
                        if s + 1 < N_STEPS:
                            descs[s + 1][sub] = mk_d(s + 1, sub)
                            descs[s + 1][sub].start()
                        out_ref[rows, :] = ag[s - N_RS, srows, :].astype(jnp.float32)

    bf = jnp.bfloat16
    return pl.pallas_call(
        body,
        out_shape=jax.ShapeDtypeStruct((m, n), jnp.float32),
        in_specs=[
            pl.BlockSpec(memory_space=pltpu.VMEM),
            pl.BlockSpec(memory_space=pltpu.VMEM),
        ],
        out_specs=pl.BlockSpec(memory_space=pltpu.VMEM),
        scratch_shapes=[
            pltpu.VMEM((N_RS, ch, n), bf),
            pltpu.VMEM((N_RS, ch, n), bf),
            pltpu.VMEM((N_RS + 1, ch, n), bf),
            pltpu.VMEM((N_RS + 1, ch, n), bf),
            pltpu.VMEM((N_RS, ch, n), bf),
            pltpu.VMEM((N_RS, ch, n), bf),
            pltpu.SemaphoreType.DMA((N_STEPS, SUBS)),
            pltpu.SemaphoreType.DMA((N_STEPS, SUBS)),
            pltpu.SemaphoreType.DMA((N_STEPS, SUBS)),
            pltpu.SemaphoreType.DMA((N_STEPS, SUBS)),
        ],
        compiler_params=pltpu.CompilerParams(collective_id=0),
    )(A, B)
